# baseline (speedup 1.0000x reference)
import jax
import jax.numpy as jnp
from jax.experimental import pallas as pl
from jax.experimental.pallas import tpu as pltpu


def _body(wi_ref, emb_hbm, mask_ref,
          qf_ref, rf_ref, wxf_ref, whf_ref, bf_ref,
          qb_ref, rb_ref, wxb_ref, whb_ref, bb_ref,
          wtf_ref, wtb_ref, bt_ref,
          out_ref, xs_ref, hf_ref, cf_ref, hb_ref, cb_ref, acc_ref, sem):
    T, BB, E = xs_ref.shape
    H = hf_ref.shape[1]
    NT = wtf_ref.shape[1]
    V = emb_hbm.shape[0]
    b0 = pl.program_id(0) * BB

    def gather_phase(emb_vmem):
        cp = pltpu.make_async_copy(emb_hbm, emb_vmem, sem)
        cp.start()
        cp.wait()

        def g_body(t, carry):
            for g0 in range(0, BB, 8):
                rows = [emb_vmem[wi_ref[b0 + g0 + j, t]] for j in range(8)]
                xs_ref[t, pl.ds(g0, 8)] = jnp.concatenate(rows, axis=0)
            return carry

        jax.lax.fori_loop(0, T, g_body, 0)

    pl.run_scoped(gather_phase, pltpu.VMEM((V, 1, E), jnp.float32))

    def compute_phase(pf_ref, pb_ref):
        def dot32(a, w):
            return jnp.dot(a.astype(jnp.bfloat16), w,
                           preferred_element_type=jnp.float32)

        def cell(x, q, r, wx, wh, b, h_ref, c_ref):
            h0 = h_ref[...]
            c0 = c_ref[...]
            xm = jax.nn.sigmoid(dot32(h0, q)) * x
            hm = jax.nn.sigmoid(dot32(xm, r)) * h0
            g = dot32(xm, wx) + dot32(hm, wh) + b
            c1 = (jax.nn.sigmoid(g[:, H:2 * H]) * c0
                  + jax.nn.sigmoid(g[:, 0:H]) * jnp.tanh(g[:, 2 * H:3 * H]))
            h1 = jax.nn.sigmoid(g[:, 3 * H:4 * H]) * jnp.tanh(c1)
            h_ref[...] = h1
            c_ref[...] = c1
            return h1

        z = jnp.zeros((BB, H), jnp.float32)
        hf_ref[...] = z
        cf_ref[...] = z
        hb_ref[...] = z
        cb_ref[...] = z

        def step(s, carry):
            tb = T - 1 - s
            for h0 in range(0, BB, BB // 2):
                sl = pl.ds(h0, BB // 2)
                h1f = cell(xs_ref[s, sl], qf_ref[...], rf_ref[...],
                           wxf_ref[...], whf_ref[...], bf_ref[...],
                           hf_ref.at[sl], cf_ref.at[sl])
                pf_ref[s, sl] = dot32(h1f, wtf_ref[...])
                h1b = cell(xs_ref[tb, sl], qb_ref[...], rb_ref[...],
                           wxb_ref[...], whb_ref[...], bb_ref[...],
                           hb_ref.at[sl], cb_ref.at[sl])
                pb_ref[tb, sl] = dot32(h1b, wtb_ref[...])
            return carry

        jax.lax.fori_loop(0, T, step, 0)

        lane_t = jax.lax.broadcasted_iota(jnp.int32, (BB, T), 1)
        lane_nt = jax.lax.broadcasted_iota(jnp.int32, (BB, NT), 1)

        def fin(t, carry):
            logits = pf_ref[t] + pb_ref[t] + bt_ref[...]
            m = jnp.max(logits, axis=1, keepdims=True)
            tag = jnp.min(jnp.where(logits == m, lane_nt, NT), axis=1,
                          keepdims=True).astype(jnp.int32)
            acc_ref[...] = jnp.where(lane_t == t, tag, acc_ref[...])
            return carry

        jax.lax.fori_loop(0, T, fin, 0)
        out_ref[...] = acc_ref[...] * mask_ref[...]

    pl.run_scoped(compute_phase,
                  pltpu.VMEM((T, BB, NT), jnp.float32),
                  pltpu.VMEM((T, BB, NT), jnp.float32))


def kernel(word_inputs, word_seq_lengths, char_inputs, batch_label, mask,
           embed, Wih_f, Whh_f, bih_f, bhh_f, Q_f, R_f,
           Wih_b, Whh_b, bih_b, bhh_b, Q_b, R_b, Wt, bt):
    B, T = word_inputs.shape
    V, E = embed.shape
    H = Whh_f.shape[1]
    NT = Wt.shape[0]
    BB = B // 2

    wi = word_inputs.astype(jnp.int32)
    maski = mask.astype(jnp.int32)
    emb3 = embed.reshape(V, 1, E)
    bf16 = jnp.bfloat16
    qf, qb = Q_f.astype(bf16), Q_b.astype(bf16)
    wxf, whf = (2.0 * Wih_f.T).astype(bf16), (2.0 * Whh_f.T).astype(bf16)
    wxb, whb = (2.0 * Wih_b.T).astype(bf16), (2.0 * Whh_b.T).astype(bf16)
    rf, rb = (2.0 * R_f).astype(bf16), (2.0 * R_b).astype(bf16)
    bf = (bih_f + bhh_f).reshape(1, 4 * H)
    bb = (bih_b + bhh_b).reshape(1, 4 * H)
    wtf = Wt[:, :H].T.astype(bf16)
    wtb = Wt[:, H:].T.astype(bf16)
    bt2 = bt.reshape(1, NT)

    def wspec(a):
        nd = a.ndim
        return pl.BlockSpec(a.shape, lambda i, _n=nd: (0,) * _n)

    weights = (qf, rf, wxf, whf, bf, qb, rb, wxb, whb, bb, wtf, wtb, bt2)
    out = pl.pallas_call(
        _body,
        grid=(2,),
        in_specs=[
            pl.BlockSpec(memory_space=pltpu.SMEM),
            pl.BlockSpec(memory_space=pl.ANY),
            pl.BlockSpec((BB, T), lambda i: (i, 0)),
        ] + [wspec(w) for w in weights],
        out_specs=pl.BlockSpec((BB, T), lambda i: (i, 0)),
        out_shape=jax.ShapeDtypeStruct((B, T), jnp.int32),
        scratch_shapes=[
            pltpu.VMEM((T, BB, E), jnp.float32),
            pltpu.VMEM((BB, H), jnp.float32),
            pltpu.VMEM((BB, H), jnp.float32),
            pltpu.VMEM((BB, H), jnp.float32),
            pltpu.VMEM((BB, H), jnp.float32),
            pltpu.VMEM((BB, T), jnp.int32),
            pltpu.SemaphoreType.DMA,
        ],
        compiler_params=pltpu.CompilerParams(
            dimension_semantics=("parallel",),
            vmem_limit_bytes=56 * 1024 * 1024,
        ),
        name="ner_bilstm_decode",
    )(wi, emb3, maski, *weights)
    return out


# speedup vs baseline: 1.2598x; 1.2598x over previous
import jax
import jax.numpy as jnp
from jax.experimental import pallas as pl
from jax.experimental.pallas import tpu as pltpu


def _body(wi_ref, emb_hbm, mask_ref,
          qf_ref, rf_ref, wxf_ref, whf_ref, bf_ref,
          qb_ref, rb_ref, wxb_ref, whb_ref, bb_ref,
          wtf_ref, wtb_ref, bt_ref,
          out_ref, xs_ref, hf_ref, cf_ref, hb_ref, cb_ref, acc_ref, sem):
    T, BB, E = xs_ref.shape
    H = hf_ref.shape[1]
    NT = wtf_ref.shape[1]
    V = emb_hbm.shape[0]
    b0 = pl.program_id(0) * BB

    def gather_phase(emb_vmem):
        cp = pltpu.make_async_copy(emb_hbm, emb_vmem, sem)
        cp.start()
        cp.wait()

        def g_body(t, carry):
            for g0 in range(0, BB, 8):
                rows = [emb_vmem[wi_ref[b0 + g0 + j, t]] for j in range(8)]
                xs_ref[t, pl.ds(g0, 8)] = jnp.concatenate(rows, axis=0)
            return carry

        jax.lax.fori_loop(0, T, g_body, 0)

    pl.run_scoped(gather_phase, pltpu.VMEM((V, 1, E), jnp.float32))

    def compute_phase(pf_ref, pb_ref):
        def dot32(a, w):
            return jnp.dot(a, w, preferred_element_type=jnp.float32)

        def cell(x, q, r, wx, wh, b, h_ref, c_ref):
            h0 = h_ref[...]
            c0 = c_ref[...]
            xm = jax.nn.sigmoid(dot32(h0, q)) * x
            hm = jax.nn.sigmoid(dot32(xm, r)) * h0
            g = dot32(xm, wx) + dot32(hm, wh) + b
            c1 = (jax.nn.sigmoid(g[:, H:2 * H]) * c0
                  + jax.nn.sigmoid(g[:, 0:H]) * jnp.tanh(g[:, 2 * H:3 * H]))
            h1 = jax.nn.sigmoid(g[:, 3 * H:4 * H]) * jnp.tanh(c1)
            h_ref[...] = h1
            c_ref[...] = c1
            return h1

        z = jnp.zeros((BB, H), jnp.float32)
        hf_ref[...] = z
        cf_ref[...] = z
        hb_ref[...] = z
        cb_ref[...] = z

        lane_t = jax.lax.broadcasted_iota(jnp.int32, (BB, T), 1)
        lane_nt = jax.lax.broadcasted_iota(jnp.int32, (BB, NT), 1)

        def decode(logits, t):
            m = jnp.max(logits, axis=1, keepdims=True)
            tag = jnp.min(jnp.where(logits == m, lane_nt, NT), axis=1,
                          keepdims=True).astype(jnp.int32)
            acc_ref[...] = jnp.where(lane_t == t, tag, acc_ref[...])

        def step(s, carry):
            h1f = cell(xs_ref[s], qf_ref[...], rf_ref[...], wxf_ref[...],
                       whf_ref[...], bf_ref[...], hf_ref, cf_ref)
            pfv = dot32(h1f, wtf_ref[...])
            pf_ref[s] = pfv
            tb = T - 1 - s
            h1b = cell(xs_ref[tb], qb_ref[...], rb_ref[...], wxb_ref[...],
                       whb_ref[...], bb_ref[...], hb_ref, cb_ref)
            pbv = dot32(h1b, wtb_ref[...])
            pb_ref[tb] = pbv

            @pl.when(s >= T // 2)
            def _():
                decode(pfv + pb_ref[s] + bt_ref[...], s)
                decode(pf_ref[tb] + pbv + bt_ref[...], tb)

            return carry

        jax.lax.fori_loop(0, T, step, 0)
        out_ref[...] = acc_ref[...] * mask_ref[...]

    pl.run_scoped(compute_phase,
                  pltpu.VMEM((T, BB, NT), jnp.float32),
                  pltpu.VMEM((T, BB, NT), jnp.float32))


def kernel(word_inputs, word_seq_lengths, char_inputs, batch_label, mask,
           embed, Wih_f, Whh_f, bih_f, bhh_f, Q_f, R_f,
           Wih_b, Whh_b, bih_b, bhh_b, Q_b, R_b, Wt, bt):
    B, T = word_inputs.shape
    V, E = embed.shape
    H = Whh_f.shape[1]
    NT = Wt.shape[0]
    BB = B // 2

    wi = word_inputs.astype(jnp.int32)
    maski = mask.astype(jnp.int32)
    emb3 = embed.reshape(V, 1, E)
    qf, qb = Q_f, Q_b
    wxf, whf = 2.0 * Wih_f.T, 2.0 * Whh_f.T
    wxb, whb = 2.0 * Wih_b.T, 2.0 * Whh_b.T
    rf, rb = 2.0 * R_f, 2.0 * R_b
    bf = (bih_f + bhh_f).reshape(1, 4 * H)
    bb = (bih_b + bhh_b).reshape(1, 4 * H)
    wtf = Wt[:, :H].T
    wtb = Wt[:, H:].T
    bt2 = bt.reshape(1, NT)

    def wspec(a):
        nd = a.ndim
        return pl.BlockSpec(a.shape, lambda i, _n=nd: (0,) * _n)

    weights = (qf, rf, wxf, whf, bf, qb, rb, wxb, whb, bb, wtf, wtb, bt2)
    out = pl.pallas_call(
        _body,
        grid=(2,),
        in_specs=[
            pl.BlockSpec(memory_space=pltpu.SMEM),
            pl.BlockSpec(memory_space=pl.ANY),
            pl.BlockSpec((BB, T), lambda i: (i, 0)),
        ] + [wspec(w) for w in weights],
        out_specs=pl.BlockSpec((BB, T), lambda i: (i, 0)),
        out_shape=jax.ShapeDtypeStruct((B, T), jnp.int32),
        scratch_shapes=[
            pltpu.VMEM((T, BB, E), jnp.float32),
            pltpu.VMEM((BB, H), jnp.float32),
            pltpu.VMEM((BB, H), jnp.float32),
            pltpu.VMEM((BB, H), jnp.float32),
            pltpu.VMEM((BB, H), jnp.float32),
            pltpu.VMEM((BB, T), jnp.int32),
            pltpu.SemaphoreType.DMA,
        ],
        compiler_params=pltpu.CompilerParams(
            dimension_semantics=("parallel",),
            vmem_limit_bytes=56 * 1024 * 1024,
        ),
        name="ner_bilstm_decode",
    )(wi, emb3, maski, *weights)
    return out
